# Initial kernel scaffold; baseline (speedup 1.0000x reference)
import jax
import jax.numpy as jnp
from jax.experimental import pallas as pl


def kernel(rank_similarity_stimulus_set, rank_similarity_is_select, percept_gate_weights_0, percept_gate_weights_1, E0, E1, E2, E3, w_minkowski):
    raise NotImplementedError("write your pallas kernel here")


# speedup vs baseline: 369.7216x; 369.7216x over previous
import functools

import jax
import jax.numpy as jnp
from jax import lax
from jax.experimental import pallas as pl
from jax.experimental.pallas import tpu as pltpu
from jax.experimental.pallas import tpu_sc as plsc

B, NCOL, NOUT = 4096, 9, 56
NREF = NCOL - 1
BETA = 10.0
EPS = 1e-7
LANES = 16
NOUTP = 64
NVEC = NOUTP // LANES
NC, NS = 2, 16
NW = NC * NS
BPW = B // NW
ROW = NCOL * NOUT
IDX_WORDS = BPW * ROW
PAD_TAIL = 64


def _sc_body(idx_hbm, g_hbm, sel_hbm, e_hbm, out_hbm,
             idx_v, g_v, sel_v, e_v, mbuf, stab, out_v):
    wid = lax.axis_index("s") * NC + lax.axis_index("c")

    pltpu.sync_copy(idx_hbm.at[pl.ds(wid * IDX_WORDS, IDX_WORDS)],
                    idx_v.at[pl.ds(0, IDX_WORDS)])
    pltpu.sync_copy(g_hbm.at[pl.ds(wid * (BPW * 4), BPW * 4)], g_v)
    pltpu.sync_copy(sel_hbm.at[pl.ds(wid * (BPW * NCOL), BPW * NCOL)], sel_v)
    pltpu.sync_copy(e_hbm, e_v)

    lanes = lax.iota(jnp.int32, LANES)
    pq0 = lax.shift_right_logical(lanes, 2)
    pr0 = lanes & 3
    pq1 = pq0 + 4
    pr1 = pr0 + 4
    mask_r0 = (pr0 != 0).astype(jnp.float32)
    zeros_i = jnp.zeros((LANES,), jnp.int32)
    ones_f = jnp.ones((LANES,), jnp.float32)

    ea = e_v[pl.ds(0, LANES)]
    eb = e_v[pl.ds(LANES, LANES)]
    ec = e_v[pl.ds(2 * LANES, LANES)]
    ed = e_v[pl.ds(3 * LANES, LANES)]

    def bstep(b, carry):
        gb = b * 4
        g00 = plsc.load_gather(g_v, [zeros_i + gb])
        g01 = plsc.load_gather(g_v, [zeros_i + (gb + 1)])
        g10 = plsc.load_gather(g_v, [zeros_i + (gb + 2)])
        g11 = plsc.load_gather(g_v, [zeros_i + (gb + 3)])
        mbuf[...] = g00 * (g10 * ea + g11 * eb) + g01 * (g10 * ec + g11 * ed)
        vq0 = plsc.load_gather(mbuf, [pq0])
        vr0 = plsc.load_gather(mbuf, [pr0])
        vq1 = plsc.load_gather(mbuf, [pq1])
        vr1 = plsc.load_gather(mbuf, [pr1])
        dz0 = vq0 - vr0
        dz1 = vq1 - vr1
        d2 = dz0 * dz0 + dz1 * dz1
        d2c = jnp.maximum(d2, 1e-30)
        yi = jnp.int32(0x5F3759DF) - lax.shift_right_logical(
            plsc.bitcast(d2c, jnp.int32), 1)
        y = plsc.bitcast(yi, jnp.float32)
        y = y * (1.5 - 0.5 * d2c * y * y)
        y = y * (1.5 - 0.5 * d2c * y * y)
        y = y * (1.5 - 0.5 * d2c * y * y)
        d = d2 * y
        stab[...] = jnp.exp(-BETA * d) * mask_r0

        selv = [plsc.load_gather(sel_v, [zeros_i + (b * NCOL + c)])
                for c in range(1, NCOL)]

        rowbase = b * ROW
        for v in range(NVEC):
            off = v * LANES
            qv = idx_v[pl.ds(rowbase + off, LANES)]
            qs = lax.shift_left(qv & 3, 2)
            denom = jnp.zeros((LANES,), jnp.float32)
            prod = ones_f
            for c in range(NREF, 0, -1):
                rv = idx_v[pl.ds(rowbase + c * NOUT + off, LANES)]
                code = qs | (rv & 3)
                sv = plsc.load_gather(stab, [code])
                denom = denom + sv
                t = 1.0 + selv[c - 1] * (sv / jnp.maximum(denom, EPS) - 1.0)
                prod = prod * t
            out_v[pl.ds(b * NOUTP + off, LANES)] = prod
        return carry

    lax.fori_loop(0, BPW, bstep, 0)
    pltpu.sync_copy(out_v, out_hbm.at[pl.ds(wid * (BPW * NOUTP), BPW * NOUTP)])


@functools.partial(
    pl.kernel,
    out_type=jax.ShapeDtypeStruct((B * NOUTP,), jnp.float32),
    mesh=plsc.VectorSubcoreMesh(core_axis_name="c", subcore_axis_name="s"),
    compiler_params=pltpu.CompilerParams(needs_layout_passes=False),
    scratch_types=[
        pltpu.VMEM((IDX_WORDS + PAD_TAIL,), jnp.int32),
        pltpu.VMEM((BPW * 4,), jnp.float32),
        pltpu.VMEM((BPW * NCOL,), jnp.float32),
        pltpu.VMEM((4 * LANES,), jnp.float32),
        pltpu.VMEM((LANES,), jnp.float32),
        pltpu.VMEM((LANES,), jnp.float32),
        pltpu.VMEM((BPW * NOUTP,), jnp.float32),
    ],
)
def _rank_model_sc(idx_hbm, g_hbm, sel_hbm, e_hbm, out_hbm,
                   idx_v, g_v, sel_v, e_v, mbuf, stab, out_v):
    _sc_body(idx_hbm, g_hbm, sel_hbm, e_hbm, out_hbm,
             idx_v, g_v, sel_v, e_v, mbuf, stab, out_v)


def kernel(rank_similarity_stimulus_set, rank_similarity_is_select,
           percept_gate_weights_0, percept_gate_weights_1,
           E0, E1, E2, E3, w_minkowski):
    idx_flat = rank_similarity_stimulus_set.reshape(B * ROW)
    gcat = jnp.concatenate(
        [percept_gate_weights_0, percept_gate_weights_1], axis=1
    ).reshape(B * 4)
    sel = rank_similarity_is_select[:, :, 0].astype(jnp.float32).reshape(B * NCOL)
    ws = jnp.sqrt(w_minkowski)

    def evec(E):
        v = (E * ws[None, :]).T.reshape(8)
        return jnp.concatenate([v, jnp.zeros((8,), jnp.float32)])

    econst = jnp.concatenate([evec(E0), evec(E1), evec(E2), evec(E3)])
    out = _rank_model_sc(idx_flat, gcat, sel, econst)
    return out.reshape(B, NOUTP)[:, :NOUT]


# speedup vs baseline: 375.7289x; 1.0162x over previous
import functools

import jax
import jax.numpy as jnp
from jax import lax
from jax.experimental import pallas as pl
from jax.experimental.pallas import tpu as pltpu
from jax.experimental.pallas import tpu_sc as plsc

B, NCOL, NOUT = 4096, 9, 56
NREF = NCOL - 1
BETA = 10.0
EPS = 1e-7
LANES = 16
NOUTP = 64
NVEC = NOUTP // LANES
NC, NS = 2, 16
NW = NC * NS
BPW = B // NW
ROW = NCOL * NOUT
IDX_WORDS = BPW * ROW
PAD_TAIL = 64


def _sc_body(idx_hbm, g_hbm, sel_hbm, e_hbm, out_hbm,
             idx_v, g_v, sel_v, e_v, mbuf, stab, out_v):
    wid = lax.axis_index("s") * NC + lax.axis_index("c")

    pltpu.sync_copy(idx_hbm.at[pl.ds(wid * IDX_WORDS, IDX_WORDS)],
                    idx_v.at[pl.ds(0, IDX_WORDS)])
    pltpu.sync_copy(g_hbm.at[pl.ds(wid * (BPW * 4), BPW * 4)], g_v)
    pltpu.sync_copy(sel_hbm.at[pl.ds(wid * (BPW * NCOL), BPW * NCOL)], sel_v)
    pltpu.sync_copy(e_hbm, e_v)

    lanes = lax.iota(jnp.int32, LANES)
    pq0 = lax.shift_right_logical(lanes, 2)
    pr0 = lanes & 3
    pq1 = pq0 + 4
    pr1 = pr0 + 4
    mask_r0 = (pr0 != 0).astype(jnp.float32)
    zeros_i = jnp.zeros((LANES,), jnp.int32)
    ones_f = jnp.ones((LANES,), jnp.float32)
    for t in range(PAD_TAIL // LANES):
        idx_v[pl.ds(IDX_WORDS + t * LANES, LANES)] = zeros_i

    ea = e_v[pl.ds(0, LANES)]
    eb = e_v[pl.ds(LANES, LANES)]
    ec = e_v[pl.ds(2 * LANES, LANES)]
    ed = e_v[pl.ds(3 * LANES, LANES)]

    def bstep(b, carry):
        gb = b * 4
        g00 = plsc.load_gather(g_v, [zeros_i + gb])
        g01 = plsc.load_gather(g_v, [zeros_i + (gb + 1)])
        g10 = plsc.load_gather(g_v, [zeros_i + (gb + 2)])
        g11 = plsc.load_gather(g_v, [zeros_i + (gb + 3)])
        mbuf[...] = g00 * (g10 * ea + g11 * eb) + g01 * (g10 * ec + g11 * ed)
        vq0 = plsc.load_gather(mbuf, [pq0])
        vr0 = plsc.load_gather(mbuf, [pr0])
        vq1 = plsc.load_gather(mbuf, [pq1])
        vr1 = plsc.load_gather(mbuf, [pr1])
        dz0 = vq0 - vr0
        dz1 = vq1 - vr1
        d2 = dz0 * dz0 + dz1 * dz1
        d2c = jnp.maximum(d2, 1e-30)
        yi = jnp.int32(0x5F3759DF) - lax.shift_right_logical(
            plsc.bitcast(d2c, jnp.int32), 1)
        y = plsc.bitcast(yi, jnp.float32)
        y = y * (1.5 - 0.5 * d2c * y * y)
        y = y * (1.5 - 0.5 * d2c * y * y)
        y = y * (1.5 - 0.5 * d2c * y * y)
        d = d2 * y
        stab[...] = jnp.exp(-BETA * d) * mask_r0

        selv = [plsc.load_gather(sel_v, [zeros_i + (b * NCOL + c)]) > 0.5
                for c in range(1, NCOL)]

        rowbase = b * ROW
        for v in range(NVEC):
            off = v * LANES
            qv = idx_v[pl.ds(rowbase + off, LANES)]
            qs = lax.shift_left(qv, 2)
            denom = jnp.zeros((LANES,), jnp.float32)
            prod = ones_f
            for half in range(2):
                num = ones_f
                den = ones_f
                for c in range(NREF - 4 * half, NREF - 4 * (half + 1), -1):
                    rv = idx_v[pl.ds(rowbase + c * NOUT + off, LANES)]
                    code = qs | rv
                    sv = plsc.load_gather(stab, [code])
                    denom = denom + sv
                    dmax = jnp.maximum(denom, EPS)
                    num = num * jnp.where(selv[c - 1], sv, 1.0)
                    den = den * jnp.where(selv[c - 1], dmax, 1.0)
                prod = prod * (num / den)
            prod = jnp.where(prod > 0.0, prod, 0.0)
            out_v[pl.ds(b * NOUTP + off, LANES)] = prod
        return carry

    lax.fori_loop(0, BPW, bstep, 0)
    pltpu.sync_copy(out_v, out_hbm.at[pl.ds(wid * (BPW * NOUTP), BPW * NOUTP)])


@functools.partial(
    pl.kernel,
    out_type=jax.ShapeDtypeStruct((B * NOUTP,), jnp.float32),
    mesh=plsc.VectorSubcoreMesh(core_axis_name="c", subcore_axis_name="s"),
    compiler_params=pltpu.CompilerParams(needs_layout_passes=False),
    scratch_types=[
        pltpu.VMEM((IDX_WORDS + PAD_TAIL,), jnp.int32),
        pltpu.VMEM((BPW * 4,), jnp.float32),
        pltpu.VMEM((BPW * NCOL,), jnp.float32),
        pltpu.VMEM((4 * LANES,), jnp.float32),
        pltpu.VMEM((LANES,), jnp.float32),
        pltpu.VMEM((LANES,), jnp.float32),
        pltpu.VMEM((BPW * NOUTP,), jnp.float32),
    ],
)
def _rank_model_sc(idx_hbm, g_hbm, sel_hbm, e_hbm, out_hbm,
                   idx_v, g_v, sel_v, e_v, mbuf, stab, out_v):
    _sc_body(idx_hbm, g_hbm, sel_hbm, e_hbm, out_hbm,
             idx_v, g_v, sel_v, e_v, mbuf, stab, out_v)


def kernel(rank_similarity_stimulus_set, rank_similarity_is_select,
           percept_gate_weights_0, percept_gate_weights_1,
           E0, E1, E2, E3, w_minkowski):
    idx_flat = rank_similarity_stimulus_set.reshape(B * ROW)
    gcat = jnp.concatenate(
        [percept_gate_weights_0, percept_gate_weights_1], axis=1
    ).reshape(B * 4)
    sel = rank_similarity_is_select[:, :, 0].astype(jnp.float32).reshape(B * NCOL)
    ws = jnp.sqrt(w_minkowski)

    def evec(E):
        v = (E * ws[None, :]).T.reshape(8)
        return jnp.concatenate([v, jnp.zeros((8,), jnp.float32)])

    econst = jnp.concatenate([evec(E0), evec(E1), evec(E2), evec(E3)])
    out = _rank_model_sc(idx_flat, gcat, sel, econst)
    return out.reshape(B, NOUTP)[:, :NOUT]
